# baseline (speedup 1.0000x reference)
import functools

import jax
import jax.numpy as jnp
from jax import lax
from jax.experimental import pallas as pl
from jax.experimental.pallas import tpu as pltpu
from jax.experimental.pallas import tpu_sc as plsc

_B = 8
_C = 64
_HW = 4096
_K = 1024
_NBLK = 4096
_GJ = _HW // _NBLK
_LOSS_SCALE = 0.25 * 10.0 / float(_B * _C * _HW)


def _tc_body(x_ref, e_ref, idx_ref, loss_ref, et_ref, em2_ref, e2_ref):
    first = (pl.program_id(0) == 0) & (pl.program_id(1) == 0)
    last = (pl.program_id(0) == _B - 1) & (pl.program_id(1) == _GJ - 1)

    @pl.when(first)
    def _():
        e = e_ref[...]
        em2_ref[...] = -2.0 * e
        e2_ref[...] = jnp.sum(e * e, axis=1, keepdims=True)
        et_ref[...] = jnp.transpose(e)
        loss_ref[0, 0] = 0.0

    x = x_ref[0]
    d = lax.dot_general(
        em2_ref[...], x, (((1,), (0,)), ((), ())),
        preferred_element_type=jnp.float32,
    ) + e2_ref[...]
    dmin = jnp.min(d, axis=0, keepdims=True)
    idx_ref[0] = jnp.argmin(d, axis=0).astype(jnp.int32)[None, :]

    loss_ref[0, 0] += jnp.sum(x * x) + jnp.sum(dmin)

    @pl.when(last)
    def _():
        loss_ref[0, 0] *= _LOSS_SCALE


def _tc_call(x3, e):
    return pl.pallas_call(
        _tc_body,
        grid=(_B, _GJ),
        in_specs=[
            pl.BlockSpec((1, _C, _NBLK), lambda b, j: (b, 0, j)),
            pl.BlockSpec((_K, _C), lambda b, j: (0, 0)),
        ],
        out_specs=[
            pl.BlockSpec((1, 1, _NBLK), lambda b, j: (b * _GJ + j, 0, 0)),
            pl.BlockSpec(memory_space=pltpu.SMEM),
            pl.BlockSpec((_C, _K), lambda b, j: (0, 0)),
        ],
        out_shape=[
            jax.ShapeDtypeStruct((_B * _GJ, 1, _NBLK), jnp.int32),
            jax.ShapeDtypeStruct((1, 1), jnp.float32),
            jax.ShapeDtypeStruct((_C, _K), jnp.float32),
        ],
        scratch_shapes=[
            pltpu.VMEM((_K, _C), jnp.float32),
            pltpu.VMEM((_K, 1), jnp.float32),
        ],
    )(x3, e)


_DPASS = 8


def _sc_gather(e_t_flat, idx):
    mesh = plsc.VectorSubcoreMesh(core_axis_name="c", subcore_axis_name="s")

    @functools.partial(
        pl.kernel,
        mesh=mesh,
        compiler_params=pltpu.CompilerParams(needs_layout_passes=False),
        out_type=jax.ShapeDtypeStruct((_B, _C, _HW), jnp.float32),
        scratch_types=[
            pltpu.VMEM((16 * _K,), jnp.float32),
            pltpu.VMEM((_HW,), jnp.int32),
            pltpu.VMEM((16, _HW), jnp.float32),
            pltpu.SemaphoreType.DMA,
        ],
    )
    def k(et_hbm, idx_hbm, out_hbm, et_v, idx_v, rows_v, sem):
        cid = lax.axis_index("c")
        sid = lax.axis_index("s")
        wid = sid * 2 + cid
        b = wid // 4
        dg = wid % 4
        pltpu.sync_copy(et_hbm.at[pl.ds(dg * 16 * _K, 16 * _K)], et_v)
        pltpu.sync_copy(idx_hbm.at[b], idx_v)

        @plsc.parallel_loop(0, _HW // 16, 1, unroll=4)
        def chunk(ci):
            iv = idx_v[pl.ds(ci * 16, 16)]
            for dl in range(16):
                vals = plsc.load_gather(et_v, [iv + dl * _K])
                rows_v[dl, pl.ds(ci * 16, 16)] = vals

        copies = [
            pltpu.async_copy(rows_v.at[dl], out_hbm.at[b, dg * 16 + dl], sem)
            for dl in range(16)
        ]
        for cp in copies:
            cp.wait()

    return k(e_t_flat, idx)


def kernel(inputs, embedding_weight):
    x3 = inputs.reshape(_B, _C, _HW)
    idx3, loss_sum, e_t = _tc_call(x3, embedding_weight)
    idx = idx3.reshape(_B, _HW)
    q = _sc_gather(e_t.reshape(_C * _K), idx)
    quantized_st = q.reshape(_B, _C, 64, 64)
    loss = loss_sum[0, 0]
    encoding_indices = idx.reshape(_B, 64, 64)
    encodings_sum = jnp.zeros(256, dtype=jnp.float32)
    return (quantized_st, loss, encoding_indices, encodings_sum, embedding_weight)


# speedup vs baseline: 1.0229x; 1.0229x over previous
import functools

import jax
import jax.numpy as jnp
from jax import lax
from jax.experimental import pallas as pl
from jax.experimental.pallas import tpu as pltpu
from jax.experimental.pallas import tpu_sc as plsc

_B = 8
_C = 64
_HW = 4096
_K = 1024
_NBLK = 4096
_GJ = _HW // _NBLK
_LOSS_SCALE = 0.25 * 10.0 / float(_B * _C * _HW)


def _tc_body(x_ref, e_ref, idx_ref, loss_ref, et_ref, em2_ref, e2_ref):
    first = (pl.program_id(0) == 0) & (pl.program_id(1) == 0)
    last = (pl.program_id(0) == _B - 1) & (pl.program_id(1) == _GJ - 1)

    @pl.when(first)
    def _():
        e = e_ref[...]
        em2_ref[...] = -2.0 * e
        e2_ref[...] = jnp.sum(e * e, axis=1, keepdims=True)
        et_ref[...] = jnp.transpose(e)
        loss_ref[0, 0] = 0.0

    x = x_ref[0].reshape(_C, _NBLK)
    d = lax.dot_general(
        em2_ref[...], x, (((1,), (0,)), ((), ())),
        preferred_element_type=jnp.float32,
    ) + e2_ref[...]
    dmin = jnp.min(d, axis=0, keepdims=True)
    idx_ref[0] = jnp.argmin(d, axis=0).astype(jnp.int32)[None, :]

    loss_ref[0, 0] += jnp.sum(x * x) + jnp.sum(dmin)

    @pl.when(last)
    def _():
        loss_ref[0, 0] *= _LOSS_SCALE


def _tc_call(x3, e):
    return pl.pallas_call(
        _tc_body,
        grid=(_B, _GJ),
        in_specs=[
            pl.BlockSpec((1, _C, 64, 64), lambda b, j: (b, 0, 0, 0)),
            pl.BlockSpec((_K, _C), lambda b, j: (0, 0)),
        ],
        out_specs=[
            pl.BlockSpec((1, 1, _NBLK), lambda b, j: (b * _GJ + j, 0, 0)),
            pl.BlockSpec(memory_space=pltpu.SMEM),
            pl.BlockSpec((_C, _K), lambda b, j: (0, 0)),
        ],
        out_shape=[
            jax.ShapeDtypeStruct((_B * _GJ, 1, _NBLK), jnp.int32),
            jax.ShapeDtypeStruct((1, 1), jnp.float32),
            jax.ShapeDtypeStruct((_C, _K), jnp.float32),
        ],
        scratch_shapes=[
            pltpu.VMEM((_K, _C), jnp.float32),
            pltpu.VMEM((_K, 1), jnp.float32),
        ],
    )(x3, e)


_DPASS = 8


def _sc_gather(e_t_flat, idx):
    mesh = plsc.VectorSubcoreMesh(core_axis_name="c", subcore_axis_name="s")

    @functools.partial(
        pl.kernel,
        mesh=mesh,
        compiler_params=pltpu.CompilerParams(needs_layout_passes=False),
        out_type=jax.ShapeDtypeStruct((_B, _C, _HW), jnp.float32),
        scratch_types=[
            pltpu.VMEM((16 * _K,), jnp.float32),
            pltpu.VMEM((_HW,), jnp.int32),
            pltpu.VMEM((16, _HW), jnp.float32),
            pltpu.SemaphoreType.DMA,
        ],
    )
    def k(et_hbm, idx_hbm, out_hbm, et_v, idx_v, rows_v, sem):
        cid = lax.axis_index("c")
        sid = lax.axis_index("s")
        wid = sid * 2 + cid
        b = wid // 4
        dg = wid % 4
        pltpu.sync_copy(et_hbm.at[pl.ds(dg * 16 * _K, 16 * _K)], et_v)
        pltpu.sync_copy(idx_hbm.at[b], idx_v)

        @plsc.parallel_loop(0, _HW // 16, 1, unroll=4)
        def chunk(ci):
            iv = idx_v[pl.ds(ci * 16, 16)]
            for dl in range(16):
                vals = plsc.load_gather(et_v, [iv + dl * _K])
                rows_v[dl, pl.ds(ci * 16, 16)] = vals

        copies = [
            pltpu.async_copy(rows_v.at[dl], out_hbm.at[b, dg * 16 + dl], sem)
            for dl in range(16)
        ]
        for cp in copies:
            cp.wait()

    return k(e_t_flat, idx)


def _relayout_body(q_ref, o_ref):
    o_ref[0] = q_ref[0].reshape(_C // 4, 64, 64)


def _relayout(q):
    return pl.pallas_call(
        _relayout_body,
        grid=(_B, 4),
        in_specs=[pl.BlockSpec((1, _C // 4, _HW), lambda b, j: (b, j, 0))],
        out_specs=pl.BlockSpec((1, _C // 4, 64, 64), lambda b, j: (b, j, 0, 0)),
        out_shape=jax.ShapeDtypeStruct((_B, _C, 64, 64), jnp.float32),
    )(q)


def kernel(inputs, embedding_weight):
    idx3, loss_sum, e_t = _tc_call(inputs, embedding_weight)
    idx = idx3.reshape(_B, _HW)
    q = _sc_gather(e_t.reshape(_C * _K), idx)
    quantized_st = _relayout(q)
    loss = loss_sum[0, 0]
    encoding_indices = idx.reshape(_B, 64, 64)
    encodings_sum = jnp.zeros(256, dtype=jnp.float32)
    return (quantized_st, loss, encoding_indices, encodings_sum, embedding_weight)


# speedup vs baseline: 1.1952x; 1.1685x over previous
import functools

import jax
import jax.numpy as jnp
from jax import lax
from jax.experimental import pallas as pl
from jax.experimental.pallas import tpu as pltpu
from jax.experimental.pallas import tpu_sc as plsc

_B = 8
_C = 64
_HW = 4096
_K = 1024
_NBLK = 4096
_GJ = _HW // _NBLK
_LOSS_SCALE = 0.25 * 10.0 / float(_B * _C * _HW)


def _tc_body(x_ref, e_ref, idx_ref, loss_ref, et_ref, em2_ref, e2_ref):
    first = (pl.program_id(0) == 0) & (pl.program_id(1) == 0)
    last = (pl.program_id(0) == _B - 1) & (pl.program_id(1) == _GJ - 1)

    @pl.when(first)
    def _():
        e = e_ref[...]
        em2_ref[...] = -2.0 * e
        e2_ref[...] = jnp.sum(e * e, axis=1, keepdims=True)
        et_ref[...] = jnp.transpose(e)
        loss_ref[0, 0] = 0.0

    x = x_ref[0].reshape(_C, _NBLK)
    d = lax.dot_general(
        em2_ref[...], x, (((1,), (0,)), ((), ())),
        preferred_element_type=jnp.float32,
    ) + e2_ref[...]
    dmin = jnp.min(d, axis=0, keepdims=True)
    idx_ref[0] = jnp.argmin(d, axis=0).astype(jnp.int32)[None, :]

    loss_ref[0, 0] += jnp.sum(x * x) + jnp.sum(dmin)

    @pl.when(last)
    def _():
        loss_ref[0, 0] *= _LOSS_SCALE


def _tc_call(x3, e):
    return pl.pallas_call(
        _tc_body,
        grid=(_B, _GJ),
        in_specs=[
            pl.BlockSpec((1, _C, 64, 64), lambda b, j: (b, 0, 0, 0)),
            pl.BlockSpec((_K, _C), lambda b, j: (0, 0)),
        ],
        out_specs=[
            pl.BlockSpec((1, 1, _NBLK), lambda b, j: (b * _GJ + j, 0, 0)),
            pl.BlockSpec(memory_space=pltpu.SMEM),
            pl.BlockSpec((_C, _K), lambda b, j: (0, 0)),
        ],
        out_shape=[
            jax.ShapeDtypeStruct((_B * _GJ, 1, _NBLK), jnp.int32),
            jax.ShapeDtypeStruct((1, 1), jnp.float32),
            jax.ShapeDtypeStruct((_C, _K), jnp.float32),
        ],
        scratch_shapes=[
            pltpu.VMEM((_K, _C), jnp.float32),
            pltpu.VMEM((_K, 1), jnp.float32),
        ],
    )(x3, e)


_DPASS = 8


def _sc_gather(e_t_flat, idx):
    mesh = plsc.VectorSubcoreMesh(core_axis_name="c", subcore_axis_name="s")

    @functools.partial(
        pl.kernel,
        mesh=mesh,
        compiler_params=pltpu.CompilerParams(needs_layout_passes=False),
        out_type=jax.ShapeDtypeStruct((_B, _C, _HW), jnp.float32),
        scratch_types=[
            pltpu.VMEM((16, _K), jnp.float32),
            pltpu.VMEM((_HW,), jnp.int32),
            pltpu.VMEM((16, _HW), jnp.float32),
            pltpu.SemaphoreType.DMA,
            pltpu.SemaphoreType.DMA,
        ],
    )
    def k(et_hbm, idx_hbm, out_hbm, et_v, idx_v, rows_v, sem, sem2):
        cid = lax.axis_index("c")
        sid = lax.axis_index("s")
        wid = sid * 2 + cid
        b = wid // 4
        dg = wid % 4
        cp_e = pltpu.async_copy(et_hbm.at[pl.ds(dg * 16, 16)], et_v, sem)
        cp_i = pltpu.async_copy(idx_hbm.at[b, 0], idx_v, sem2)
        cp_e.wait()
        cp_i.wait()

        @plsc.parallel_loop(0, _HW // 16, 1, unroll=4)
        def chunk(ci):
            iv = idx_v[pl.ds(ci * 16, 16)]
            for dl in range(16):
                vals = plsc.load_gather(et_v, [jnp.zeros((16,), jnp.int32) + dl, iv])
                rows_v[dl, pl.ds(ci * 16, 16)] = vals

        copies = [
            pltpu.async_copy(rows_v.at[dl], out_hbm.at[b, dg * 16 + dl], sem)
            for dl in range(16)
        ]
        for cp in copies:
            cp.wait()

    return k(e_t_flat, idx)


def kernel(inputs, embedding_weight):
    idx3, loss_sum, e_t = _tc_call(inputs, embedding_weight)
    q = _sc_gather(e_t, idx3)
    quantized_st = q.reshape(_B, _C, 64, 64)
    loss = loss_sum[0, 0]
    encoding_indices = idx3.reshape(_B, 64, 64)
    encodings_sum = jnp.zeros(256, dtype=jnp.float32)
    return (quantized_st, loss, encoding_indices, encodings_sum, embedding_weight)
